# baseline (speedup 1.0000x reference)
import functools

import jax
import jax.numpy as jnp
from jax import lax
from jax.experimental import pallas as pl
from jax.experimental.pallas import tpu as pltpu
from jax.experimental.pallas import tpu_sc as plsc

VOCAB = 1000000
DIM = 64
BATCH = 16384
HIST = 50
B = BATCH * HIST

_info = plsc.get_sparse_core_info()
NC = _info.num_cores
NS = _info.num_subcores
NW = NC * NS

CH = 256
B_PER_W = B // NW
NROWS = B_PER_W // CH
K = 2
SB = K * CH
NSC = NROWS // K
NCOL = B // 128

_mesh = plsc.VectorSubcoreMesh(core_axis_name="c", subcore_axis_name="s")


@functools.partial(
    pl.kernel,
    mesh=_mesh,
    out_type=jax.ShapeDtypeStruct((8, NCOL, 1024), jnp.float32),
    scratch_types=[
        pltpu.VMEM((NROWS, CH), jnp.int32),
        pltpu.VMEM((SB, DIM), jnp.float32),
        pltpu.VMEM((SB, DIM), jnp.float32),
        pltpu.VMEM((2, 8, 1024), jnp.float32),
        pltpu.SemaphoreType.DMA,
        pltpu.SemaphoreType.DMA,
        pltpu.SemaphoreType.DMA,
        pltpu.SemaphoreType.DMA,
    ],
    compiler_params=pltpu.CompilerParams(
        use_tc_tiling_on_sc=False, needs_layout_passes=False),
)
def _embed_lookup(idx_hbm, table_hbm, out_hbm, idx_v, buf0, buf1, bufT,
                  sem0, sem1, ssem0, ssem1):
    wid = lax.axis_index("s") * NC + lax.axis_index("c")
    base = wid * B_PER_W

    pltpu.sync_copy(idx_hbm.at[wid], idx_v)

    iota16 = lax.iota(jnp.int32, 16)
    pos_of_t = ((iota16 >> 3) << 10) + ((iota16 & 7) << 7)

    def issue(buf, sem, sc):
        for k in range(K):
            pltpu.async_copy(
                table_hbm.at[idx_v.at[sc * K + k]],
                buf.at[pl.ds(k * CH, CH)],
                sem,
            )

    def drain_gather(buf, sem):
        pltpu.make_async_copy(table_hbm.at[pl.ds(0, SB)], buf, sem).wait()

    issue(buf0, sem0, 0)
    issue(buf1, sem1, 1)

    def transpose_chunk(buf, j, t):
        dst = bufT.at[t, 0]

        def body(lo, pvec):
            for dl in range(8):
                l = lo * 8 + dl
                for g in range(4):
                    x = buf[j * 128 + l, pl.ds(g * 16, 16)]
                    plsc.store_scatter(dst, [pvec + (dl + g * 2048)], x)
            return pvec + 8

        lax.fori_loop(0, 16, body, pos_of_t)

    def outer(o, carry):
        for p, (buf, sem) in enumerate(((buf0, sem0), (buf1, sem1))):
            sc = o * 2 + p
            drain_gather(buf, sem)
            col0 = (base + sc * SB) // 128
            for j in range(SB // 128):
                t = j % 2
                ssem = ssem0 if t == 0 else ssem1
                q = sc * (SB // 128) + j

                @pl.when(q >= 2)
                def _():
                    pltpu.make_async_copy(
                        bufT.at[t], out_hbm.at[:, 0], ssem).wait()

                transpose_chunk(buf, j, t)
                pltpu.async_copy(bufT.at[t], out_hbm.at[:, col0 + j], ssem)

            @pl.when(sc + 2 < NSC)
            def _():
                issue(buf, sem, sc + 2)

        return carry

    lax.fori_loop(0, NSC // 2, outer, 0)

    pltpu.make_async_copy(bufT.at[0], out_hbm.at[:, 0], ssem0).wait()
    pltpu.make_async_copy(bufT.at[1], out_hbm.at[:, 0], ssem1).wait()


def kernel(tokens, table):
    idx3 = tokens.reshape(NW, NROWS, CH)
    out4 = _embed_lookup(idx3, table).reshape(8, NCOL, 8, 128)
    return out4.transpose(1, 3, 0, 2).reshape(B, DIM)
